# baseline (speedup 1.0000x reference)
import functools

import jax
import jax.numpy as jnp
from jax import lax
from jax.experimental import pallas as pl
from jax.experimental.pallas import tpu as pltpu
from jax.experimental.pallas import tpu_sc as plsc

N = 10000
E = 320000
D = 128

CHUNK = 128
NC = 2
NS = 16
BLK = 16
T0 = 32
T1 = 128
CHUNKS_TOTAL = NS * (T0 + T1)
E_PAD = CHUNKS_TOTAL * CHUNK
ACC_ROWS = 10240
RPT = ACC_ROWS // NS


def _sc_aggregate(x, src2d, dst2d, w2d):
    mesh = plsc.VectorSubcoreMesh(core_axis_name="c", subcore_axis_name="s")

    @functools.partial(
        pl.kernel,
        out_type=jax.ShapeDtypeStruct((NC, ACC_ROWS, D), jnp.float32),
        mesh=mesh,
        scratch_types=[
            pltpu.VMEM((2, BLK, CHUNK), jnp.int32),
            pltpu.VMEM((2, BLK, CHUNK), jnp.int32),
            pltpu.VMEM((2, BLK, CHUNK), jnp.float32),
            pltpu.VMEM((2, CHUNK, D), jnp.float32),
            pltpu.SemaphoreType.DMA((2,)),
            pltpu.SemaphoreType.DMA((2,)),
            pltpu.SemaphoreType.DMA((2,)),
            pltpu.VMEM_SHARED((ACC_ROWS, D), jnp.float32),
        ],
    )
    def k(x_hbm, src_hbm, dst_hbm, w_hbm, out_hbm,
          src_blk, dst_blk, w_blk, rows, gsem, ssem, isem, acc):
        cid = lax.axis_index("c")
        sid = lax.axis_index("s")

        def zrow(r, carry):
            for c in range(D // 16):
                rows[0, r, pl.ds(c * 16, 16)] = jnp.zeros((16,), jnp.float32)
            return carry

        lax.fori_loop(0, CHUNK, zrow, 0)
        for q in range(RPT // CHUNK):
            pltpu.sync_copy(rows.at[0],
                            acc.at[pl.ds(sid * RPT + q * CHUNK, CHUNK)])

        nblk = jnp.where(cid == 0, T0 // BLK, T1 // BLK)
        wbase = pl.multiple_of(
            jnp.where(cid == 0, sid * T0, NS * T0 + sid * T1), 8)

        def start_idx_load(kb, p):
            hb = pl.multiple_of(wbase + kb * BLK, 8)
            pltpu.async_copy(src_hbm.at[pl.ds(hb, BLK)], src_blk.at[p],
                             isem.at[p])
            pltpu.async_copy(dst_hbm.at[pl.ds(hb, BLK)], dst_blk.at[p],
                             isem.at[p])
            pltpu.async_copy(w_hbm.at[pl.ds(hb, BLK)], w_blk.at[p],
                             isem.at[p])

        def wait_idx_load(kb, p):
            hb = pl.multiple_of(wbase + kb * BLK, 8)
            pltpu.make_async_copy(src_hbm.at[pl.ds(hb, BLK)], src_blk.at[p],
                                  isem.at[p]).wait()
            pltpu.make_async_copy(dst_hbm.at[pl.ds(hb, BLK)], dst_blk.at[p],
                                  isem.at[p]).wait()
            pltpu.make_async_copy(w_hbm.at[pl.ds(hb, BLK)], w_blk.at[p],
                                  isem.at[p]).wait()

        start_idx_load(0, 0)
        plsc.subcore_barrier()

        def block_body(kb, carry):
            p = lax.rem(kb, 2)
            wait_idx_load(kb, p)
            pltpu.async_copy(x_hbm.at[src_blk.at[p, 0]], rows.at[0],
                             gsem.at[0])

            @pl.when(kb + 1 < nblk)
            def _():
                start_idx_load(kb + 1, 1 - p)

            def pair_body(t, carry2):
                for b in range(2):
                    jj = t * 2 + b
                    j = kb * BLK + jj
                    pltpu.make_async_copy(x_hbm.at[src_blk.at[p, jj]],
                                          rows.at[b], gsem.at[b]).wait()

                    @pl.when(j >= 1)
                    def _():
                        pltpu.make_async_copy(rows.at[1 - b],
                                              acc.at[dst_blk.at[p, jj]],
                                              ssem.at[1 - b]).wait()

                    @pl.when(jj + 1 < BLK)
                    def _():
                        pltpu.async_copy(x_hbm.at[src_blk.at[p, jj + 1]],
                                         rows.at[1 - b], gsem.at[1 - b])

                    def group_body(g, c2):
                        wv = w_blk[p, jj, pl.ds(g * 16, 16)]
                        for e2 in range(16):
                            ws = wv[e2]
                            row_e = g * 16 + e2
                            for c in range(D // 16):
                                sl = pl.ds(c * 16, 16)
                                rows[b, row_e, sl] = rows[b, row_e, sl] * ws
                        return c2

                    lax.fori_loop(0, CHUNK // 16, group_body, 0)
                    pltpu.async_copy(rows.at[b], acc.at[dst_blk.at[p, jj]],
                                     ssem.at[b], add=True)
                return carry2

            lax.fori_loop(0, BLK // 2, pair_body, 0)
            return carry

        lax.fori_loop(0, nblk, block_body, 0)
        pltpu.make_async_copy(rows.at[1], acc.at[dst_blk.at[0, BLK - 1]],
                              ssem.at[1]).wait()
        plsc.subcore_barrier()
        pltpu.sync_copy(acc.at[pl.ds(sid * RPT, RPT)],
                        out_hbm.at[cid, pl.ds(sid * RPT, RPT)])

    return k(x, src2d, dst2d, w2d)


def _tc_combine(p0, p1, W):
    BM = 2000

    def body(p0_ref, p1_ref, w_ref, o_ref):
        s = p0_ref[...] + p1_ref[...]
        o_ref[...] = jnp.maximum(
            jnp.dot(s, w_ref[...], preferred_element_type=jnp.float32), 0.0)

    return pl.pallas_call(
        body,
        grid=(N // BM,),
        in_specs=[
            pl.BlockSpec((BM, D), lambda i: (i, 0)),
            pl.BlockSpec((BM, D), lambda i: (i, 0)),
            pl.BlockSpec((D, D), lambda i: (0, 0)),
        ],
        out_specs=pl.BlockSpec((BM, D), lambda i: (i, 0)),
        out_shape=jax.ShapeDtypeStruct((N, D), jnp.float32),
    )(p0, p1, W)


@jax.jit
def kernel(x, edge_index, edge_weight, W):
    pad = E_PAD - E
    src = jnp.concatenate([edge_index[1], jnp.zeros((pad,), jnp.int32)])
    dst = jnp.concatenate(
        [edge_index[0],
         N + (jnp.arange(pad, dtype=jnp.int32) % (ACC_ROWS - N))])
    w = jnp.concatenate([edge_weight, jnp.zeros((pad,), jnp.float32)])
    src2d = src.reshape(CHUNKS_TOTAL, CHUNK)
    dst2d = dst.reshape(CHUNKS_TOTAL, CHUNK)
    w2d = w.reshape(CHUNKS_TOTAL, CHUNK)
    partials = _sc_aggregate(x, src2d, dst2d, w2d)
    return _tc_combine(partials[0, :N], partials[1, :N], W)


# speedup vs baseline: 1.1962x; 1.1962x over previous
import functools

import jax
import jax.numpy as jnp
from jax import lax
from jax.experimental import pallas as pl
from jax.experimental.pallas import tpu as pltpu
from jax.experimental.pallas import tpu_sc as plsc

N = 10000
E = 320000
D = 128

CHUNK = 128
NC = 2
NS = 16
NW = NC * NS
CHUNKS_TOTAL = -(-E // (CHUNK * NW)) * NW
CPW = CHUNKS_TOTAL // NW
E_PAD = CHUNKS_TOTAL * CHUNK
ACC_ROWS = 10240
RPT = ACC_ROWS // NS


def _sc_aggregate(x, src2d, dst2d, w2d):
    mesh = plsc.VectorSubcoreMesh(core_axis_name="c", subcore_axis_name="s")

    @functools.partial(
        pl.kernel,
        out_type=jax.ShapeDtypeStruct((NC, ACC_ROWS, D), jnp.float32),
        mesh=mesh,
        scratch_types=[
            pltpu.VMEM((CHUNK,), jnp.int32),
            pltpu.VMEM((CHUNK,), jnp.int32),
            pltpu.VMEM((CHUNK,), jnp.float32),
            pltpu.VMEM((CHUNK, D), jnp.float32),
            pltpu.VMEM((CHUNK, D), jnp.float32),
            pltpu.VMEM_SHARED((ACC_ROWS, D), jnp.float32),
            pltpu.SemaphoreType.DMA,
        ],
    )
    def k(x_hbm, src_hbm, dst_hbm, w_hbm, out_hbm,
          src_v, dst_v, w_v, rows_v, zbuf, acc, sem):
        cid = lax.axis_index("c")
        sid = lax.axis_index("s")
        wid = cid * NS + sid

        def zrow(r, carry):
            for c in range(D // 16):
                zbuf[r, pl.ds(c * 16, 16)] = jnp.zeros((16,), jnp.float32)
            return carry

        lax.fori_loop(0, CHUNK, zrow, 0)
        for q in range(RPT // CHUNK):
            pltpu.sync_copy(zbuf, acc.at[pl.ds(sid * RPT + q * CHUNK, CHUNK)])
        plsc.subcore_barrier()

        def chunk_body(j, carry):
            row = wid * CPW + j
            pltpu.sync_copy(src_hbm.at[row], src_v)
            pltpu.sync_copy(dst_hbm.at[row], dst_v)
            pltpu.sync_copy(w_hbm.at[row], w_v)
            pltpu.async_copy(x_hbm.at[src_v], rows_v, sem).wait()

            def group_body(g, c2):
                wv = w_v[pl.ds(g * 16, 16)]
                for e2 in range(16):
                    ws = wv[e2]
                    row_e = g * 16 + e2
                    for c in range(D // 16):
                        sl = pl.ds(c * 16, 16)
                        rows_v[row_e, sl] = rows_v[row_e, sl] * ws
                return c2

            lax.fori_loop(0, CHUNK // 16, group_body, 0)
            pltpu.sync_copy(rows_v, acc.at[dst_v], add=True)
            return carry

        lax.fori_loop(0, CPW, chunk_body, 0)
        plsc.subcore_barrier()
        pltpu.sync_copy(acc.at[pl.ds(sid * RPT, RPT)],
                        out_hbm.at[cid, pl.ds(sid * RPT, RPT)])

    return k(x, src2d, dst2d, w2d)


def _tc_combine(p0, p1, W):
    BM = 2000

    def body(p0_ref, p1_ref, w_ref, o_ref):
        s = p0_ref[...] + p1_ref[...]
        o_ref[...] = jnp.maximum(
            jnp.dot(s, w_ref[...], preferred_element_type=jnp.float32), 0.0)

    return pl.pallas_call(
        body,
        grid=(N // BM,),
        in_specs=[
            pl.BlockSpec((BM, D), lambda i: (i, 0)),
            pl.BlockSpec((BM, D), lambda i: (i, 0)),
            pl.BlockSpec((D, D), lambda i: (0, 0)),
        ],
        out_specs=pl.BlockSpec((BM, D), lambda i: (i, 0)),
        out_shape=jax.ShapeDtypeStruct((N, D), jnp.float32),
    )(p0, p1, W)


@jax.jit
def kernel(x, edge_index, edge_weight, W):
    pad = E_PAD - E
    src = jnp.concatenate([edge_index[1], jnp.zeros((pad,), jnp.int32)])
    dst = jnp.concatenate(
        [edge_index[0],
         N + (jnp.arange(pad, dtype=jnp.int32) % (ACC_ROWS - N))])
    w = jnp.concatenate([edge_weight, jnp.zeros((pad,), jnp.float32)])
    src2d = src.reshape(CHUNKS_TOTAL, CHUNK)
    dst2d = dst.reshape(CHUNKS_TOTAL, CHUNK)
    w2d = w.reshape(CHUNKS_TOTAL, CHUNK)
    partials = _sc_aggregate(x, src2d, dst2d, w2d)
    return _tc_combine(partials[0, :N], partials[1, :N], W)


# speedup vs baseline: 1.1982x; 1.0017x over previous
import functools

import jax
import jax.numpy as jnp
from jax import lax
from jax.experimental import pallas as pl
from jax.experimental.pallas import tpu as pltpu
from jax.experimental.pallas import tpu_sc as plsc

N = 10000
E = 320000
D = 128

CHUNK = 128
NC = 2
NS = 16
NW = NC * NS
CHUNKS_TOTAL = -(-E // (CHUNK * NW)) * NW
CPW = CHUNKS_TOTAL // NW
E_PAD = CHUNKS_TOTAL * CHUNK
ACC_ROWS = 10240
RPT = ACC_ROWS // NS


def _sc_aggregate(x, src2d, dst2d, w2d):
    mesh = plsc.VectorSubcoreMesh(core_axis_name="c", subcore_axis_name="s")

    @functools.partial(
        pl.kernel,
        out_type=jax.ShapeDtypeStruct((NC, ACC_ROWS, D), jnp.float32),
        mesh=mesh,
        scratch_types=[
            pltpu.VMEM((CHUNK,), jnp.int32),
            pltpu.VMEM((CHUNK,), jnp.int32),
            pltpu.VMEM((CHUNK,), jnp.float32),
            pltpu.VMEM((CHUNK, D), jnp.float32),
            pltpu.VMEM((CHUNK, D), jnp.float32),
            pltpu.VMEM_SHARED((ACC_ROWS, D), jnp.float32),
            pltpu.SemaphoreType.DMA,
            pltpu.SemaphoreType.DMA,
        ],
    )
    def k(x_hbm, src_hbm, dst_hbm, w_hbm, out_hbm,
          src_v, dst_v, w_v, rows_v, zbuf, acc, sem, sem2):
        cid = lax.axis_index("c")
        sid = lax.axis_index("s")
        wid = cid * NS + sid

        def zrow(r, carry):
            for c in range(D // 16):
                zbuf[r, pl.ds(c * 16, 16)] = jnp.zeros((16,), jnp.float32)
            return carry

        lax.fori_loop(0, CHUNK, zrow, 0)
        for q in range(RPT // CHUNK):
            pltpu.sync_copy(zbuf, acc.at[pl.ds(sid * RPT + q * CHUNK, CHUNK)])
        plsc.subcore_barrier()

        def chunk_body(j, carry):
            row = wid * CPW + j
            pltpu.sync_copy(src_hbm.at[row], src_v)
            pltpu.sync_copy(dst_hbm.at[row], dst_v)
            pltpu.sync_copy(w_hbm.at[row], w_v)
            h0 = pltpu.async_copy(x_hbm.at[src_v.at[pl.ds(0, CHUNK // 2)]],
                                  rows_v.at[pl.ds(0, CHUNK // 2)], sem)
            h1 = pltpu.async_copy(
                x_hbm.at[src_v.at[pl.ds(CHUNK // 2, CHUNK // 2)]],
                rows_v.at[pl.ds(CHUNK // 2, CHUNK // 2)], sem2)
            h0.wait()
            h1.wait()

            def group_body(g, c2):
                wv = w_v[pl.ds(g * 16, 16)]
                for e2 in range(16):
                    ws = wv[e2]
                    row_e = g * 16 + e2
                    for c in range(D // 16):
                        sl = pl.ds(c * 16, 16)
                        rows_v[row_e, sl] = rows_v[row_e, sl] * ws
                return c2

            lax.fori_loop(0, CHUNK // 16, group_body, 0)
            pltpu.sync_copy(rows_v, acc.at[dst_v], add=True)
            return carry

        lax.fori_loop(0, CPW, chunk_body, 0)
        plsc.subcore_barrier()
        pltpu.sync_copy(acc.at[pl.ds(sid * RPT, RPT)],
                        out_hbm.at[cid, pl.ds(sid * RPT, RPT)])

    return k(x, src2d, dst2d, w2d)


def _tc_combine(p0, p1, W):
    BM = 2000

    def body(p0_ref, p1_ref, w_ref, o_ref):
        s = p0_ref[...] + p1_ref[...]
        o_ref[...] = jnp.maximum(
            jnp.dot(s, w_ref[...], preferred_element_type=jnp.float32), 0.0)

    return pl.pallas_call(
        body,
        grid=(N // BM,),
        in_specs=[
            pl.BlockSpec((BM, D), lambda i: (i, 0)),
            pl.BlockSpec((BM, D), lambda i: (i, 0)),
            pl.BlockSpec((D, D), lambda i: (0, 0)),
        ],
        out_specs=pl.BlockSpec((BM, D), lambda i: (i, 0)),
        out_shape=jax.ShapeDtypeStruct((N, D), jnp.float32),
    )(p0, p1, W)


@jax.jit
def kernel(x, edge_index, edge_weight, W):
    pad = E_PAD - E
    src = jnp.concatenate([edge_index[1], jnp.zeros((pad,), jnp.int32)])
    dst = jnp.concatenate(
        [edge_index[0],
         N + (jnp.arange(pad, dtype=jnp.int32) % (ACC_ROWS - N))])
    w = jnp.concatenate([edge_weight, jnp.zeros((pad,), jnp.float32)])
    src2d = src.reshape(CHUNKS_TOTAL, CHUNK)
    dst2d = dst.reshape(CHUNKS_TOTAL, CHUNK)
    w2d = w.reshape(CHUNKS_TOTAL, CHUNK)
    partials = _sc_aggregate(x, src2d, dst2d, w2d)
    return _tc_combine(partials[0, :N], partials[1, :N], W)
